# baseline (speedup 1.0000x reference)
import functools

import jax
import jax.numpy as jnp
from jax import lax
from jax.experimental import pallas as pl
from jax.experimental.pallas import tpu as pltpu
from jax.experimental.pallas import tpu_sc as plsc


@functools.lru_cache(maxsize=None)
def _build(B: int, V: int, D: int):
    info = plsc.get_sparse_core_info()
    NC, NS, L = info.num_cores, info.num_subcores, info.num_lanes
    NW = NC * NS
    assert B % NW == 0 and (B // NW) % 8 == 0 and D % L == 0
    b_per_w = B // NW
    mesh = plsc.VectorSubcoreMesh(core_axis_name="c", subcore_axis_name="s")

    @functools.partial(
        pl.kernel,
        mesh=mesh,
        out_type=jax.ShapeDtypeStruct((B, D), jnp.float32),
        scratch_types=[
            pltpu.VMEM((b_per_w,), jnp.int32),
            pltpu.VMEM((b_per_w, D), jnp.float32),
            pltpu.SemaphoreType.DMA,
        ],
    )
    def k(idx_hbm, table_hbm, out_hbm, idx_v, rows_v, sem):
        wid = lax.axis_index("s") * NC + lax.axis_index("c")
        base = wid * b_per_w
        pltpu.sync_copy(idx_hbm.at[pl.ds(base, b_per_w)], idx_v)
        lanes = lax.iota(jnp.int32, L)
        copies = []
        for g in range(b_per_w // L):
            ids16 = idx_v[pl.ds(g * L, L)]
            for l in range(L):
                r = ids16[l]
                i = g * L + l
                copies.append(
                    pltpu.async_copy(table_hbm.at[r], rows_v.at[i], sem))
        for c in copies:
            c.wait()
        pltpu.sync_copy(rows_v, out_hbm.at[pl.ds(base, b_per_w)])

    return k


def kernel(input_ids, embedding):
    B = input_ids.shape[0]
    V, D = embedding.shape
    ids = input_ids.astype(jnp.int32)
    table = embedding.astype(jnp.float32)
    return _build(B, V, D)(ids, table)


# speedup vs baseline: 1.4892x; 1.4892x over previous
import functools

import jax
import jax.numpy as jnp
from jax import lax
from jax.experimental import pallas as pl
from jax.experimental.pallas import tpu as pltpu
from jax.experimental.pallas import tpu_sc as plsc

_NBUF = 4


@functools.lru_cache(maxsize=None)
def _build(B: int, V: int, D: int):
    info = plsc.get_sparse_core_info()
    NC, NS, L = info.num_cores, info.num_subcores, info.num_lanes
    NW = NC * NS
    assert B % NW == 0 and (B // NW) % 8 == 0 and D % L == 0
    b_per_w = B // NW
    mesh = plsc.VectorSubcoreMesh(core_axis_name="c", subcore_axis_name="s")

    @functools.partial(
        pl.kernel,
        mesh=mesh,
        compiler_params=pltpu.CompilerParams(needs_layout_passes=False),
        out_type=jax.ShapeDtypeStruct((B, D), jnp.float32),
        scratch_types=[
            pltpu.VMEM((b_per_w,), jnp.int32),
            pltpu.VMEM((_NBUF, D, 128), jnp.float32),
            pltpu.VMEM((b_per_w, D), jnp.float32),
            pltpu.SemaphoreType.DMA,
            pltpu.SemaphoreType.DMA,
            pltpu.SemaphoreType.DMA,
            pltpu.SemaphoreType.DMA,
        ],
    )
    def k(idx_hbm, tablet_hbm, out_hbm, idx_v, bufs, rows_v, *sems):
        wid = lax.axis_index("s") * NC + lax.axis_index("c")
        base = wid * b_per_w
        pltpu.sync_copy(idx_hbm.at[pl.ds(base, b_per_w)], idx_v)
        rs, offs = [], []
        for g in range(b_per_w // L):
            ids16 = idx_v[pl.ds(g * L, L)]
            for l in range(L):
                r = ids16[l]
                rs.append(r)
                offs.append(r & 127)

        def fetch(i):
            col0 = pl.multiple_of((rs[i] >> 7) * 128, 128)
            return pltpu.async_copy(
                tablet_hbm.at[:, pl.ds(col0, 128)], bufs.at[i % _NBUF],
                sems[i % _NBUF])

        copies = {}
        for i in range(_NBUF):
            copies[i] = fetch(i)
        for i in range(b_per_w):
            copies[i].wait()
            col16 = jnp.full((L,), 0, jnp.int32) + offs[i]
            for kk in range(D // L):
                row16 = lax.iota(jnp.int32, L) + kk * L
                v = plsc.load_gather(bufs.at[i % _NBUF], [row16, col16])
                rows_v[i, pl.ds(kk * L, L)] = v
            if i + _NBUF < b_per_w:
                copies[i + _NBUF] = fetch(i + _NBUF)
        pltpu.sync_copy(rows_v, out_hbm.at[pl.ds(base, b_per_w)])

    return k


def kernel(input_ids, embedding):
    B = input_ids.shape[0]
    V, D = embedding.shape
    ids = input_ids.astype(jnp.int32)
    tablet = embedding.astype(jnp.float32).T
    return _build(B, V, D)(ids, tablet)


# speedup vs baseline: 1.5500x; 1.0408x over previous
import functools

import jax
import jax.numpy as jnp
from jax import lax
from jax.experimental import pallas as pl
from jax.experimental.pallas import tpu as pltpu
from jax.experimental.pallas import tpu_sc as plsc

_NBUF = 6
_W = 128


@functools.lru_cache(maxsize=None)
def _build(B: int, V: int, D: int):
    info = plsc.get_sparse_core_info()
    NC, NS, L = info.num_cores, info.num_subcores, info.num_lanes
    NW = NC * NS
    assert B % NW == 0 and (B // NW) % 8 == 0 and D % L == 0
    b_per_w = B // NW
    mesh = plsc.VectorSubcoreMesh(core_axis_name="c", subcore_axis_name="s")

    @functools.partial(
        pl.kernel,
        mesh=mesh,
        compiler_params=pltpu.CompilerParams(needs_layout_passes=False),
        out_type=jax.ShapeDtypeStruct((B, D), jnp.float32),
        scratch_types=[
            pltpu.VMEM((b_per_w,), jnp.int32),
            pltpu.VMEM((_NBUF, D, _W), jnp.float32),
            pltpu.VMEM((b_per_w, D), jnp.float32),
        ] + [pltpu.SemaphoreType.DMA] * _NBUF,
    )
    def k(idx_hbm, tablet_hbm, out_hbm, idx_v, bufs, rows_v, *sems):
        wid = lax.axis_index("s") * NC + lax.axis_index("c")
        base = wid * b_per_w
        pltpu.sync_copy(idx_hbm.at[pl.ds(base, b_per_w)], idx_v)
        rs, offs = [], []
        for g in range(b_per_w // L):
            ids16 = idx_v[pl.ds(g * L, L)]
            for l in range(L):
                r = ids16[l]
                rs.append(r)
                offs.append(r & (_W - 1))

        def fetch(i):
            col0 = pl.multiple_of((rs[i] // _W) * _W, 128)
            return pltpu.async_copy(
                tablet_hbm.at[:, pl.ds(col0, _W)], bufs.at[i % _NBUF],
                sems[i % _NBUF])

        copies = {}
        for i in range(_NBUF):
            copies[i] = fetch(i)
        for i in range(b_per_w):
            copies[i].wait()
            col16 = jnp.full((L,), 0, jnp.int32) + offs[i]
            for kk in range(D // L):
                row16 = lax.iota(jnp.int32, L) + kk * L
                v = plsc.load_gather(bufs.at[i % _NBUF], [row16, col16])
                rows_v[i, pl.ds(kk * L, L)] = v
            if i + _NBUF < b_per_w:
                copies[i + _NBUF] = fetch(i + _NBUF)
        pltpu.sync_copy(rows_v, out_hbm.at[pl.ds(base, b_per_w)])

    return k


def kernel(input_ids, embedding):
    B = input_ids.shape[0]
    V, D = embedding.shape
    ids = input_ids.astype(jnp.int32)
    tablet = embedding.astype(jnp.float32).T
    return _build(B, V, D)(ids, tablet)


# speedup vs baseline: 1.5798x; 1.0192x over previous
import functools

import jax
import jax.numpy as jnp
from jax import lax
from jax.experimental import pallas as pl
from jax.experimental.pallas import tpu as pltpu
from jax.experimental.pallas import tpu_sc as plsc

_NBUF = 8
_W = 128


@functools.lru_cache(maxsize=None)
def _build(B: int, V: int, D: int):
    info = plsc.get_sparse_core_info()
    NC, NS, L = info.num_cores, info.num_subcores, info.num_lanes
    NW = NC * NS
    assert B % NW == 0 and (B // NW) % 8 == 0 and D % L == 0
    b_per_w = B // NW
    mesh = plsc.VectorSubcoreMesh(core_axis_name="c", subcore_axis_name="s")

    @functools.partial(
        pl.kernel,
        mesh=mesh,
        compiler_params=pltpu.CompilerParams(needs_layout_passes=False),
        out_type=jax.ShapeDtypeStruct((B, D), jnp.float32),
        scratch_types=[
            pltpu.VMEM((b_per_w,), jnp.int32),
            pltpu.VMEM((_NBUF, D, _W), jnp.float32),
            pltpu.VMEM((b_per_w, D), jnp.float32),
        ] + [pltpu.SemaphoreType.DMA] * _NBUF,
    )
    def k(idx_hbm, tablet_hbm, out_hbm, idx_v, bufs, rows_v, *sems):
        wid = lax.axis_index("s") * NC + lax.axis_index("c")
        base = wid * b_per_w
        pltpu.sync_copy(idx_hbm.at[pl.ds(base, b_per_w)], idx_v)
        rs, offs = [], []
        for g in range(b_per_w // L):
            ids16 = idx_v[pl.ds(g * L, L)]
            for l in range(L):
                r = ids16[l]
                rs.append(r)
                offs.append(r & (_W - 1))

        def fetch(i):
            col0 = pl.multiple_of((rs[i] // _W) * _W, 128)
            return pltpu.async_copy(
                tablet_hbm.at[:, pl.ds(col0, _W)], bufs.at[i % _NBUF],
                sems[i % _NBUF])

        copies = {}
        for i in range(_NBUF):
            copies[i] = fetch(i)
        for i in range(b_per_w):
            copies[i].wait()
            col16 = jnp.full((L,), 0, jnp.int32) + offs[i]
            for kk in range(D // L):
                row16 = lax.iota(jnp.int32, L) + kk * L
                v = plsc.load_gather(bufs.at[i % _NBUF], [row16, col16])
                rows_v[i, pl.ds(kk * L, L)] = v
            if i + _NBUF < b_per_w:
                copies[i + _NBUF] = fetch(i + _NBUF)
        pltpu.sync_copy(rows_v, out_hbm.at[pl.ds(base, b_per_w)])

    return k


def kernel(input_ids, embedding):
    B = input_ids.shape[0]
    V, D = embedding.shape
    ids = input_ids.astype(jnp.int32)
    tablet = embedding.astype(jnp.float32).T
    return _build(B, V, D)(ids, tablet)
